# Initial kernel scaffold; baseline (speedup 1.0000x reference)
import jax
import jax.numpy as jnp
from jax.experimental import pallas as pl


def kernel(x, A_indices, A_values, A_shape, W, bias):
    raise NotImplementedError("write your pallas kernel here")


# speedup vs baseline: 4.3318x; 4.3318x over previous
import functools

import jax
import jax.numpy as jnp
from jax import lax
from jax.experimental import pallas as pl
from jax.experimental.pallas import tpu as pltpu
from jax.experimental.pallas import tpu_sc as plsc

_NC = 2
_NS = 16
_NW = _NC * _NS
_B = 128
_L = 16


def _sc_aggregate(x, cols_r, rows_r, vals_r, n_pad):
    d = x.shape[1]
    nb = cols_r.shape[1]
    ngrp = d // _L
    rpt = n_pad // _NS
    mesh = plsc.VectorSubcoreMesh(core_axis_name="c", subcore_axis_name="s")

    @functools.partial(
        pl.kernel,
        mesh=mesh,
        out_type=jax.ShapeDtypeStruct((_NC, n_pad, d), jnp.float32),
        scratch_types=[
            pltpu.VMEM((nb, _B), jnp.int32),
            pltpu.VMEM((nb, _B), jnp.int32),
            pltpu.VMEM((nb, _B), jnp.float32),
            pltpu.VMEM((_B, d), jnp.float32),
            pltpu.VMEM_SHARED((n_pad, d), jnp.float32),
            pltpu.SemaphoreType.DMA,
        ],
    )
    def k(x_hbm, cols_hbm, rows_hbm, vals_hbm, out_hbm,
          cols_v, rows_v, vals_v, gbuf, acc, sem):
        c = lax.axis_index("c")
        s = lax.axis_index("s")
        w = c * _NS + s

        zero_row = jnp.zeros((_L,), jnp.float32)

        def zero_body(i, carry):
            for j in range(ngrp):
                gbuf[i, pl.ds(j * _L, _L)] = zero_row
            return carry

        lax.fori_loop(0, _B, zero_body, 0)
        base = s * rpt
        for blk in range(rpt // _B):
            pltpu.sync_copy(gbuf, acc.at[pl.ds(base + blk * _B, _B)])

        pltpu.sync_copy(cols_hbm.at[w], cols_v)
        pltpu.sync_copy(rows_hbm.at[w], rows_v)
        pltpu.sync_copy(vals_hbm.at[w], vals_v)
        plsc.subcore_barrier()

        def batch_body(b, carry):
            pltpu.async_copy(x_hbm.at[cols_v.at[b]], gbuf, sem).wait()

            def scale_body(g, carry2):
                vv = vals_v[b, pl.ds(g * _L, _L)]
                for l in range(_L):
                    v = vv[l]
                    i = g * _L + l
                    for j in range(ngrp):
                        sl = pl.ds(j * _L, _L)
                        gbuf[i, sl] = gbuf[i, sl] * v
                return carry2

            lax.fori_loop(0, _B // _L, scale_body, 0)

            pltpu.sync_copy(gbuf, acc.at[rows_v.at[b]], add=True)
            return carry

        lax.fori_loop(0, nb, batch_body, 0)

        plsc.subcore_barrier()
        pltpu.sync_copy(acc.at[pl.ds(base, rpt)], out_hbm.at[c, pl.ds(base, rpt)])

    return k(x, cols_r, rows_r, vals_r)


def _tc_transform(p0, p1, w_mat, bias_row):
    m, d = p0.shape
    bm = 1024

    def body(p0_ref, p1_ref, w_ref, b_ref, o_ref):
        agg = p0_ref[...] + p1_ref[...]
        h = lax.dot_general(agg, w_ref[...], (((1,), (1,)), ((), ())),
                            preferred_element_type=jnp.float32)
        o_ref[...] = jnp.maximum(h + b_ref[...], 0.0)

    return pl.pallas_call(
        body,
        grid=(m // bm,),
        in_specs=[
            pl.BlockSpec((bm, d), lambda i: (i, 0)),
            pl.BlockSpec((bm, d), lambda i: (i, 0)),
            pl.BlockSpec((d, d), lambda i: (0, 0)),
            pl.BlockSpec((1, d), lambda i: (0, 0)),
        ],
        out_specs=pl.BlockSpec((bm, d), lambda i: (i, 0)),
        out_shape=jax.ShapeDtypeStruct((m, d), jnp.float32),
    )(p0, p1, w_mat, bias_row)


def kernel(x, A_indices, A_values, A_shape, W, bias):
    n, d = x.shape
    e = A_values.shape[0]

    chunk = _NW * _B
    e_pad = ((e + chunk - 1) // chunk) * chunk
    nb = e_pad // chunk
    pad = e_pad - e
    rows = jnp.pad(A_indices[0], (0, pad)).reshape(_NW, nb, _B)
    cols = jnp.pad(A_indices[1], (0, pad)).reshape(_NW, nb, _B)
    vals = jnp.pad(A_values, (0, pad)).reshape(_NW, nb, _B)

    tile_rows = _NS * _B
    n_pad = ((n + tile_rows - 1) // tile_rows) * tile_rows

    partial = _sc_aggregate(x, cols, rows, vals, n_pad)

    residual = (jnp.asarray(A_shape) - n).astype(jnp.float32)
    bias_row = (bias + residual).reshape(1, d)
    out_full = _tc_transform(partial[0], partial[1], W, bias_row)
    return out_full[:n]
